# baseline (speedup 1.0000x reference)
import functools

import jax
import jax.numpy as jnp
from jax.experimental import pallas as pl
from jax.experimental.pallas import tpu as pltpu

EPS = 1e-6


def _dot(a, b, dims):
    return jax.lax.dot_general(a, b, (dims, ((), ())),
                               preferred_element_type=jnp.float32)


def _fused_kernel(nbatch, nb, bn, d, n_rows,
                  x_ref, y_ref, wqk_ref, bqk_ref, wqky_ref, bqky_ref,
                  wv_ref, bv_ref, wvy_ref, bvy_ref, s_ref,
                  fx_ref, fy_ref,
                  qx_s, qy_s, m0x_s, m0y_s, ks_s, kys_s, xs_s, ys_s,
                  ext_xx_s, ext_xy_s, ext_yy_s, ext_yx_s, kp_s):
    g = pl.program_id(0)
    j = pl.program_id(1)
    cur = jax.lax.rem(g, 2)
    prv = 1 - cur
    nf = jnp.float32(nb * bn)

    @pl.when(jnp.logical_and(g < nbatch, j == 0))
    def _():
        m0x_s[cur] = jnp.zeros_like(m0x_s[cur])
        m0y_s[cur] = jnp.zeros_like(m0y_s[cur])
        ks_s[cur] = jnp.zeros_like(ks_s[cur])
        kys_s[cur] = jnp.zeros_like(kys_s[cur])
        xs_s[cur] = jnp.zeros_like(xs_s[cur])
        ys_s[cur] = jnp.zeros_like(ys_s[cur])
        lane = jax.lax.broadcasted_iota(jnp.int32, (n_rows, d), 1)
        pat = jnp.where(lane == 0, 1.0, 0.0).astype(jnp.float32)
        qx_s[cur, :, d:2 * d] = pat
        qy_s[cur, :, d:2 * d] = pat

    @pl.when(g < nbatch)
    def _phase_a():
        rows = pl.ds(j * bn, bn)

        def stream(t_ref, wqk, bqk, q_s, m0_s, ks, xs):
            t = t_ref[0]
            wq = wqk[0:d, :]
            wk = wqk[d:2 * d, :]
            q = _dot(t, wq, ((1,), (1,))) + bqk[0:1, 0:d]
            k = _dot(t, wk, ((1,), (1,))) + bqk[0:1, d:2 * d]
            q = q * jax.lax.rsqrt(jnp.sum(q * q, axis=1, keepdims=True))
            k = k * jax.lax.rsqrt(jnp.sum(k * k, axis=1, keepdims=True))
            q_s[cur, rows, 0:d] = q
            m0_s[cur] += _dot(k, t, ((0,), (0,)))
            ks[cur] += jnp.sum(k, axis=0, keepdims=True)
            xs[cur] += jnp.sum(t, axis=0, keepdims=True)

        stream(x_ref, wqk_ref, bqk_ref, qx_s, m0x_s, ks_s, xs_s)
        stream(y_ref, wqky_ref, bqky_ref, qy_s, m0y_s, kys_s, ys_s)

    @pl.when(jnp.logical_and(g > 0, j == 0))
    def _():
        ksT = jnp.transpose(ks_s[prv], (1, 0))
        kysT = jnp.transpose(kys_s[prv], (1, 0))
        matx = _dot(m0x_s[prv], wv_ref[...], ((1,), (1,))) \
            + ksT * bv_ref[...]
        maty = _dot(m0y_s[prv], wvy_ref[...], ((1,), (1,))) \
            + kysT * bvy_ref[...]
        vs = _dot(xs_s[prv], wv_ref[...], ((1,), (1,))) + nf * bv_ref[...]
        vys = _dot(ys_s[prv], wvy_ref[...], ((1,), (1,))) + nf * bvy_ref[...]
        zero_tail = jnp.zeros((d - 1, matx.shape[1]), jnp.float32)
        ext_xx_s[...] = jnp.concatenate([matx, vs, zero_tail], axis=0)
        ext_xy_s[...] = jnp.concatenate([maty, vs, zero_tail], axis=0)
        ext_yy_s[...] = jnp.concatenate([maty, vys, zero_tail], axis=0)
        ext_yx_s[...] = jnp.concatenate([matx, vys, zero_tail], axis=0)
        zcol = jnp.zeros((d, 2), jnp.float32)
        kp_s[...] = jnp.concatenate(
            [jnp.concatenate([ksT + EPS, kysT + EPS], axis=1), zcol], axis=0)

    @pl.when(g > 0)
    def _phase_b():
        rows = pl.ds(j * bn, bn)
        qx = qx_s[prv, rows, :]
        qy = qy_s[prv, rows, :]
        tq = _dot(qx, kp_s[...], ((1,), (0,)))
        tqy = _dot(qy, kp_s[...], ((1,), (0,)))
        ax1 = s_ref[0] / (nf + tq[:, 0:1])
        ax2 = s_ref[1] / (nf + tq[:, 1:2])
        ay1 = s_ref[2] / (nf + tqy[:, 1:2])
        ay2 = s_ref[3] / (nf + tqy[:, 0:1])
        fx_ref[0] = _dot(ax1 * qx, ext_xx_s[...], ((1,), (0,))) \
            + _dot(ax2 * qx, ext_xy_s[...], ((1,), (0,)))
        fy_ref[0] = _dot(ay1 * qy, ext_yy_s[...], ((1,), (0,))) \
            + _dot(ay2 * qy, ext_yx_s[...], ((1,), (0,)))


def _run(x, y, Wq, bq, Wk, bk, Wv, bv, Wqy, bqy, Wky, bky, Wvy, bvy,
         gamma, gamma_y, gamma_cx, gamma_cy, wx1, wx2, wy1, wy2,
         interpret=False):
    b, n, c = x.shape
    d = Wq.shape[0]
    bn = min(2048, n)
    nb = n // bn
    f32 = jnp.float32

    in_map = lambda g, j: (jnp.where(g < b, g, b - 1),
                           jnp.where(g < b, j, nb - 1), 0)
    out_map = lambda g, j: (jnp.maximum(g - 1, 0),
                            jnp.where(g > 0, j, 0), 0)
    row_in = pl.BlockSpec((1, bn, c), in_map)
    row_out = pl.BlockSpec((1, bn, c), out_map)
    w_spec = lambda r, cc: pl.BlockSpec((r, cc), lambda g, j: (0, 0))

    wqk = jnp.concatenate([Wq, Wk], axis=0)
    wqky = jnp.concatenate([Wqy, Wky], axis=0)
    bqk = jnp.concatenate([bq, bk]).reshape(1, 2 * d)
    bqky = jnp.concatenate([bqy, bky]).reshape(1, 2 * d)
    s = jnp.stack([gamma[0] * wx1, gamma_cx[0] * wx2,
                   gamma_y[0] * wy1, gamma_cy[0] * wy2]).astype(f32)

    fx, fy = pl.pallas_call(
        functools.partial(_fused_kernel, b, nb, bn, d, n),
        grid=(b + 1, nb),
        in_specs=[
            row_in, row_in,
            w_spec(2 * d, c), w_spec(1, 2 * d),
            w_spec(2 * d, c), w_spec(1, 2 * d),
            w_spec(c, c), w_spec(1, c),
            w_spec(c, c), w_spec(1, c),
            pl.BlockSpec(memory_space=pltpu.SMEM),
        ],
        out_specs=[row_out, row_out],
        out_shape=[
            jax.ShapeDtypeStruct((b, n, c), f32),
            jax.ShapeDtypeStruct((b, n, c), f32),
        ],
        scratch_shapes=[
            pltpu.VMEM((2, n, 2 * d), f32),
            pltpu.VMEM((2, n, 2 * d), f32),
            pltpu.VMEM((2, d, c), f32),
            pltpu.VMEM((2, d, c), f32),
            pltpu.VMEM((2, 1, d), f32),
            pltpu.VMEM((2, 1, d), f32),
            pltpu.VMEM((2, 1, c), f32),
            pltpu.VMEM((2, 1, c), f32),
            pltpu.VMEM((2 * d, c), f32),
            pltpu.VMEM((2 * d, c), f32),
            pltpu.VMEM((2 * d, c), f32),
            pltpu.VMEM((2 * d, c), f32),
            pltpu.VMEM((2 * d, 2), f32),
        ],
        compiler_params=pltpu.CompilerParams(
            dimension_semantics=("arbitrary", "arbitrary"),
            vmem_limit_bytes=56 * 1024 * 1024),
        name="linattn_fused",
        interpret=interpret,
    )(x, y, wqk, bqk, wqky, bqky,
      Wv, bv.reshape(1, c), Wvy, bvy.reshape(1, c), s)
    return fx, fy


def kernel(x, y, Wq, bq, Wk, bk, Wv, bv, Wqy, bqy, Wky, bky, Wvy, bvy,
           gamma, gamma_y, gamma_cx, gamma_cy, wx1, wx2, wy1, wy2):
    return _run(x, y, Wq, bq, Wk, bk, Wv, bv, Wqy, bqy, Wky, bky, Wvy, bvy,
                gamma, gamma_y, gamma_cx, gamma_cy, wx1, wx2, wy1, wy2)


# speedup vs baseline: 1.0847x; 1.0847x over previous
import functools

import jax
import jax.numpy as jnp
from jax.experimental import pallas as pl
from jax.experimental.pallas import tpu as pltpu

EPS = 1e-6


def _dot(a, b, dims):
    return jax.lax.dot_general(a, b, (dims, ((), ())),
                               preferred_element_type=jnp.float32)


def _fused_kernel(nbatch, nb, bn, d, n_rows,
                  x_ref, y_ref, wqk_ref, bqk_ref, wqky_ref, bqky_ref,
                  wv_ref, bv_ref, wvy_ref, bvy_ref, s_ref,
                  fx_ref, fy_ref,
                  qx_s, qy_s, m0x_s, m0y_s, ks_s, kys_s, xs_s, ys_s,
                  matx_s, maty_s, vs_s, vys_s, kp_s):
    g = pl.program_id(0)
    j = pl.program_id(1)
    cur = jax.lax.rem(g, 2)
    prv = 1 - cur
    nf = jnp.float32(nb * bn)

    @pl.when(jnp.logical_and(g < nbatch, j == 0))
    def _():
        m0x_s[cur] = jnp.zeros_like(m0x_s[cur])
        m0y_s[cur] = jnp.zeros_like(m0y_s[cur])
        ks_s[cur] = jnp.zeros_like(ks_s[cur])
        kys_s[cur] = jnp.zeros_like(kys_s[cur])
        xs_s[cur] = jnp.zeros_like(xs_s[cur])
        ys_s[cur] = jnp.zeros_like(ys_s[cur])

    @pl.when(g < nbatch)
    def _phase_a():
        rows = pl.ds(j * bn, bn)

        def stream(t_ref, wqk, bqk, q_s, m0_s, ks, xs):
            t = t_ref[0]
            tb = t.astype(jnp.bfloat16)
            wq = wqk[0:d, :]
            wk = wqk[d:2 * d, :]
            q = _dot(tb, wq, ((1,), (1,))) + bqk[0:1, 0:d]
            k = _dot(tb, wk, ((1,), (1,))) + bqk[0:1, d:2 * d]
            q = q * jax.lax.rsqrt(jnp.sum(q * q, axis=1, keepdims=True))
            k = k * jax.lax.rsqrt(jnp.sum(k * k, axis=1, keepdims=True))
            q_s[cur, rows, :] = q.astype(jnp.bfloat16)
            m0_s[cur] += _dot(k.astype(jnp.bfloat16), tb, ((0,), (0,)))
            ks[cur] += jnp.sum(k, axis=0, keepdims=True)
            xs[cur] += jnp.sum(t, axis=0, keepdims=True)

        stream(x_ref, wqk_ref, bqk_ref, qx_s, m0x_s, ks_s, xs_s)
        stream(y_ref, wqky_ref, bqky_ref, qy_s, m0y_s, kys_s, ys_s)

    @pl.when(jnp.logical_and(g > 0, j == 0))
    def _():
        ksT = jnp.transpose(ks_s[prv], (1, 0))
        kysT = jnp.transpose(kys_s[prv], (1, 0))
        matx_s[...] = (_dot(m0x_s[prv], wv_ref[...], ((1,), (1,)))
                       + ksT * bv_ref[...]).astype(jnp.bfloat16)
        maty_s[...] = (_dot(m0y_s[prv], wvy_ref[...], ((1,), (1,)))
                       + kysT * bvy_ref[...]).astype(jnp.bfloat16)
        vs_s[...] = _dot(xs_s[prv], wv_ref[...], ((1,), (1,))) + nf * bv_ref[...]
        vys_s[...] = _dot(ys_s[prv], wvy_ref[...], ((1,), (1,))) + nf * bvy_ref[...]
        kp_s[:, 0:1] = (ksT + EPS).astype(jnp.bfloat16)
        kp_s[:, 1:2] = (kysT + EPS).astype(jnp.bfloat16)

    @pl.when(g > 0)
    def _phase_b():
        rows = pl.ds(j * bn, bn)
        qx = qx_s[prv, rows, :]
        qy = qy_s[prv, rows, :]
        tq = _dot(qx, kp_s[...], ((1,), (0,)))
        tqy = _dot(qy, kp_s[...], ((1,), (0,)))
        ax1 = s_ref[0] / (nf + tq[:, 0:1])
        ax2 = s_ref[1] / (nf + tq[:, 1:2])
        ay1 = s_ref[2] / (nf + tqy[:, 1:2])
        ay2 = s_ref[3] / (nf + tqy[:, 0:1])
        qm_x1 = _dot(qx, matx_s[...], ((1,), (0,)))
        qm_x2 = _dot(qx, maty_s[...], ((1,), (0,)))
        qm_y1 = _dot(qy, maty_s[...], ((1,), (0,)))
        qm_y2 = _dot(qy, matx_s[...], ((1,), (0,)))
        vs = vs_s[...]
        vys = vys_s[...]
        fx_ref[0] = ax1 * (vs + qm_x1) + ax2 * (vs + qm_x2)
        fy_ref[0] = ay1 * (vys + qm_y1) + ay2 * (vys + qm_y2)


def _run(x, y, Wq, bq, Wk, bk, Wv, bv, Wqy, bqy, Wky, bky, Wvy, bvy,
         gamma, gamma_y, gamma_cx, gamma_cy, wx1, wx2, wy1, wy2,
         interpret=False):
    b, n, c = x.shape
    d = Wq.shape[0]
    bn = min(2048, n)
    nb = n // bn
    f32 = jnp.float32

    in_map = lambda g, j: (jnp.where(g < b, g, b - 1),
                           jnp.where(g < b, j, nb - 1), 0)
    out_map = lambda g, j: (jnp.maximum(g - 1, 0),
                            jnp.where(g > 0, j, 0), 0)
    row_in = pl.BlockSpec((1, bn, c), in_map)
    row_out = pl.BlockSpec((1, bn, c), out_map)
    w_spec = lambda r, cc: pl.BlockSpec((r, cc), lambda g, j: (0, 0))

    wqk = jnp.concatenate([Wq, Wk], axis=0).astype(jnp.bfloat16)
    wqky = jnp.concatenate([Wqy, Wky], axis=0).astype(jnp.bfloat16)
    bqk = jnp.concatenate([bq, bk]).reshape(1, 2 * d)
    bqky = jnp.concatenate([bqy, bky]).reshape(1, 2 * d)
    s = jnp.stack([gamma[0] * wx1, gamma_cx[0] * wx2,
                   gamma_y[0] * wy1, gamma_cy[0] * wy2]).astype(f32)

    fx, fy = pl.pallas_call(
        functools.partial(_fused_kernel, b, nb, bn, d, n),
        grid=(b + 1, nb),
        in_specs=[
            row_in, row_in,
            w_spec(2 * d, c), w_spec(1, 2 * d),
            w_spec(2 * d, c), w_spec(1, 2 * d),
            w_spec(c, c), w_spec(1, c),
            w_spec(c, c), w_spec(1, c),
            pl.BlockSpec(memory_space=pltpu.SMEM),
        ],
        out_specs=[row_out, row_out],
        out_shape=[
            jax.ShapeDtypeStruct((b, n, c), f32),
            jax.ShapeDtypeStruct((b, n, c), f32),
        ],
        scratch_shapes=[
            pltpu.VMEM((2, n, d), jnp.bfloat16),
            pltpu.VMEM((2, n, d), jnp.bfloat16),
            pltpu.VMEM((2, d, c), f32),
            pltpu.VMEM((2, d, c), f32),
            pltpu.VMEM((2, 1, d), f32),
            pltpu.VMEM((2, 1, d), f32),
            pltpu.VMEM((2, 1, c), f32),
            pltpu.VMEM((2, 1, c), f32),
            pltpu.VMEM((d, c), jnp.bfloat16),
            pltpu.VMEM((d, c), jnp.bfloat16),
            pltpu.VMEM((1, c), f32),
            pltpu.VMEM((1, c), f32),
            pltpu.VMEM((d, 2), jnp.bfloat16),
        ],
        compiler_params=pltpu.CompilerParams(
            dimension_semantics=("arbitrary", "arbitrary"),
            vmem_limit_bytes=56 * 1024 * 1024),
        name="linattn_fused",
        interpret=interpret,
    )(x, y, wqk, bqk, wqky, bqky,
      Wv, bv.reshape(1, c), Wvy, bvy.reshape(1, c), s)
    return fx, fy


def kernel(x, y, Wq, bq, Wk, bk, Wv, bv, Wqy, bqy, Wky, bky, Wvy, bvy,
           gamma, gamma_y, gamma_cx, gamma_cy, wx1, wx2, wy1, wy2):
    return _run(x, y, Wq, bq, Wk, bk, Wv, bv, Wqy, bqy, Wky, bky, Wvy, bvy,
                gamma, gamma_y, gamma_cx, gamma_cy, wx1, wx2, wy1, wy2)


# speedup vs baseline: 1.1010x; 1.0150x over previous
import functools

import jax
import jax.numpy as jnp
from jax.experimental import pallas as pl
from jax.experimental.pallas import tpu as pltpu

EPS = 1e-6


def _dot(a, b, dims):
    return jax.lax.dot_general(a, b, (dims, ((), ())),
                               preferred_element_type=jnp.float32)


def _fused_kernel(nbatch, nb, bn, d, n_rows,
                  x_ref, y_ref, wqk_ref, bqk_ref, wqky_ref, bqky_ref,
                  wv_ref, bv_ref, wvy_ref, bvy_ref, s_ref,
                  fx_ref, fy_ref,
                  qx_s, qy_s, m0x_s, m0y_s, ks_s, kys_s, xs_s, ys_s,
                  matx_s, maty_s, vs_s, vys_s, kp_s):
    g = pl.program_id(0)
    j = pl.program_id(1)
    cur = jax.lax.rem(g, 2)
    prv = 1 - cur
    nf = jnp.float32(nb * bn)

    @pl.when(jnp.logical_and(g < nbatch, j == 0))
    def _():
        m0x_s[cur] = jnp.zeros_like(m0x_s[cur])
        m0y_s[cur] = jnp.zeros_like(m0y_s[cur])
        ks_s[cur] = jnp.zeros_like(ks_s[cur])
        kys_s[cur] = jnp.zeros_like(kys_s[cur])
        xs_s[cur] = jnp.zeros_like(xs_s[cur])
        ys_s[cur] = jnp.zeros_like(ys_s[cur])

    @pl.when(g < nbatch)
    def _phase_a():
        rows = pl.ds(j * bn, bn)

        def stream(t_ref, wqk, bqk, q_s, m0_s, ks, xs):
            t = t_ref[0]
            wq = wqk[0:d, :]
            wk = wqk[d:2 * d, :]
            q = _dot(t, wq, ((1,), (1,))) + bqk[0:1, 0:d]
            k = _dot(t, wk, ((1,), (1,))) + bqk[0:1, d:2 * d]
            q = q * jax.lax.rsqrt(jnp.sum(q * q, axis=1, keepdims=True))
            k = k * jax.lax.rsqrt(jnp.sum(k * k, axis=1, keepdims=True))
            q_s[cur, rows, :] = q
            m0_s[cur] += _dot(k, t, ((0,), (0,)))
            ks[cur] += jnp.sum(k, axis=0, keepdims=True)
            xs[cur] += jnp.sum(t, axis=0, keepdims=True)

        stream(x_ref, wqk_ref, bqk_ref, qx_s, m0x_s, ks_s, xs_s)
        stream(y_ref, wqky_ref, bqky_ref, qy_s, m0y_s, kys_s, ys_s)

    @pl.when(jnp.logical_and(g > 0, j == 0))
    def _():
        ksT = jnp.transpose(ks_s[prv], (1, 0))
        kysT = jnp.transpose(kys_s[prv], (1, 0))
        matx_s[...] = _dot(m0x_s[prv], wv_ref[...], ((1,), (1,))) \
            + ksT * bv_ref[...]
        maty_s[...] = _dot(m0y_s[prv], wvy_ref[...], ((1,), (1,))) \
            + kysT * bvy_ref[...]
        vs_s[...] = _dot(xs_s[prv], wv_ref[...], ((1,), (1,))) + nf * bv_ref[...]
        vys_s[...] = _dot(ys_s[prv], wvy_ref[...], ((1,), (1,))) + nf * bvy_ref[...]
        kp_s[:, 0:1] = ksT + EPS
        kp_s[:, 1:2] = kysT + EPS

    @pl.when(g > 0)
    def _phase_b():
        rows = pl.ds(j * bn, bn)
        qx = qx_s[prv, rows, :]
        qy = qy_s[prv, rows, :]
        tq = _dot(qx, kp_s[...], ((1,), (0,)))
        tqy = _dot(qy, kp_s[...], ((1,), (0,)))
        ax1 = s_ref[0] / (nf + tq[:, 0:1])
        ax2 = s_ref[1] / (nf + tq[:, 1:2])
        ay1 = s_ref[2] / (nf + tqy[:, 1:2])
        ay2 = s_ref[3] / (nf + tqy[:, 0:1])
        qm_x1 = _dot(qx, matx_s[...], ((1,), (0,)))
        qm_x2 = _dot(qx, maty_s[...], ((1,), (0,)))
        qm_y1 = _dot(qy, maty_s[...], ((1,), (0,)))
        qm_y2 = _dot(qy, matx_s[...], ((1,), (0,)))
        vs = vs_s[...]
        vys = vys_s[...]
        fx_ref[0] = ax1 * (vs + qm_x1) + ax2 * (vs + qm_x2)
        fy_ref[0] = ay1 * (vys + qm_y1) + ay2 * (vys + qm_y2)


def _run(x, y, Wq, bq, Wk, bk, Wv, bv, Wqy, bqy, Wky, bky, Wvy, bvy,
         gamma, gamma_y, gamma_cx, gamma_cy, wx1, wx2, wy1, wy2,
         interpret=False):
    b, n, c = x.shape
    d = Wq.shape[0]
    bn = min(2048, n)
    nb = n // bn
    f32 = jnp.float32

    in_map = lambda g, j: (jnp.where(g < b, g, b - 1),
                           jnp.where(g < b, j, nb - 1), 0)
    out_map = lambda g, j: (jnp.maximum(g - 1, 0),
                            jnp.where(g > 0, j, 0), 0)
    row_in = pl.BlockSpec((1, bn, c), in_map)
    row_out = pl.BlockSpec((1, bn, c), out_map)
    w_spec = lambda r, cc: pl.BlockSpec((r, cc), lambda g, j: (0, 0))

    wqk = jnp.concatenate([Wq, Wk], axis=0)
    wqky = jnp.concatenate([Wqy, Wky], axis=0)
    bqk = jnp.concatenate([bq, bk]).reshape(1, 2 * d)
    bqky = jnp.concatenate([bqy, bky]).reshape(1, 2 * d)
    s = jnp.stack([gamma[0] * wx1, gamma_cx[0] * wx2,
                   gamma_y[0] * wy1, gamma_cy[0] * wy2]).astype(f32)

    fx, fy = pl.pallas_call(
        functools.partial(_fused_kernel, b, nb, bn, d, n),
        grid=(b + 1, nb),
        in_specs=[
            row_in, row_in,
            w_spec(2 * d, c), w_spec(1, 2 * d),
            w_spec(2 * d, c), w_spec(1, 2 * d),
            w_spec(c, c), w_spec(1, c),
            w_spec(c, c), w_spec(1, c),
            pl.BlockSpec(memory_space=pltpu.SMEM),
        ],
        out_specs=[row_out, row_out],
        out_shape=[
            jax.ShapeDtypeStruct((b, n, c), f32),
            jax.ShapeDtypeStruct((b, n, c), f32),
        ],
        scratch_shapes=[
            pltpu.VMEM((2, n, d), f32),
            pltpu.VMEM((2, n, d), f32),
            pltpu.VMEM((2, d, c), f32),
            pltpu.VMEM((2, d, c), f32),
            pltpu.VMEM((2, 1, d), f32),
            pltpu.VMEM((2, 1, d), f32),
            pltpu.VMEM((2, 1, c), f32),
            pltpu.VMEM((2, 1, c), f32),
            pltpu.VMEM((d, c), f32),
            pltpu.VMEM((d, c), f32),
            pltpu.VMEM((1, c), f32),
            pltpu.VMEM((1, c), f32),
            pltpu.VMEM((d, 2), f32),
        ],
        compiler_params=pltpu.CompilerParams(
            dimension_semantics=("arbitrary", "arbitrary"),
            vmem_limit_bytes=56 * 1024 * 1024),
        name="linattn_fused",
        interpret=interpret,
    )(x, y, wqk, bqk, wqky, bqky,
      Wv, bv.reshape(1, c), Wvy, bvy.reshape(1, c), s)
    return fx, fy


def kernel(x, y, Wq, bq, Wk, bk, Wv, bv, Wqy, bqy, Wky, bky, Wvy, bvy,
           gamma, gamma_y, gamma_cx, gamma_cy, wx1, wx2, wy1, wy2):
    return _run(x, y, Wq, bq, Wk, bk, Wv, bv, Wqy, bqy, Wky, bky, Wvy, bvy,
                gamma, gamma_y, gamma_cx, gamma_cy, wx1, wx2, wy1, wy2)


# speedup vs baseline: 1.1019x; 1.0008x over previous
import functools

import jax
import jax.numpy as jnp
from jax.experimental import pallas as pl
from jax.experimental.pallas import tpu as pltpu

EPS = 1e-6


def _dot(a, b, dims):
    return jax.lax.dot_general(a, b, (dims, ((), ())),
                               preferred_element_type=jnp.float32)


def _fused_kernel(nbatch, nb, bn, d, n_rows,
                  x_ref, y_ref, wqk_ref, bqk_ref, wqky_ref, bqky_ref,
                  wv_ref, bv_ref, wvy_ref, bvy_ref, s_ref,
                  fx_ref, fy_ref,
                  qx_s, qy_s, m0x_s, m0y_s, ks_s, kys_s, xs_s, ys_s,
                  mats_x_s, mats_y_s, vs_s, vys_s, kp_s):
    g = pl.program_id(0)
    j = pl.program_id(1)
    cur = jax.lax.rem(g, 2)
    prv = 1 - cur
    nf = jnp.float32(nb * bn)

    @pl.when(jnp.logical_and(g < nbatch, j == 0))
    def _():
        m0x_s[cur] = jnp.zeros_like(m0x_s[cur])
        m0y_s[cur] = jnp.zeros_like(m0y_s[cur])
        ks_s[cur] = jnp.zeros_like(ks_s[cur])
        kys_s[cur] = jnp.zeros_like(kys_s[cur])
        xs_s[cur] = jnp.zeros_like(xs_s[cur])
        ys_s[cur] = jnp.zeros_like(ys_s[cur])

    @pl.when(g < nbatch)
    def _phase_a():
        rows = pl.ds(j * bn, bn)

        def stream(t_ref, wqk, bqk, q_s, m0_s, ks, xs):
            t = t_ref[0]
            wq = wqk[0:d, :]
            wk = wqk[d:2 * d, :]
            q = _dot(t, wq, ((1,), (1,))) + bqk[0:1, 0:d]
            k = _dot(t, wk, ((1,), (1,))) + bqk[0:1, d:2 * d]
            q = q * jax.lax.rsqrt(jnp.sum(q * q, axis=1, keepdims=True))
            k = k * jax.lax.rsqrt(jnp.sum(k * k, axis=1, keepdims=True))
            q_s[cur, rows, :] = q
            m0_s[cur] += _dot(k, t, ((0,), (0,)))
            ks[cur] += jnp.sum(k, axis=0, keepdims=True)
            xs[cur] += jnp.sum(t, axis=0, keepdims=True)

        stream(x_ref, wqk_ref, bqk_ref, qx_s, m0x_s, ks_s, xs_s)
        stream(y_ref, wqky_ref, bqky_ref, qy_s, m0y_s, kys_s, ys_s)

    @pl.when(jnp.logical_and(g > 0, j == 0))
    def _():
        ksT = jnp.transpose(ks_s[prv], (1, 0))
        kysT = jnp.transpose(kys_s[prv], (1, 0))
        matx = _dot(m0x_s[prv], wv_ref[...], ((1,), (1,))) \
            + ksT * bv_ref[...]
        maty = _dot(m0y_s[prv], wvy_ref[...], ((1,), (1,))) \
            + kysT * bvy_ref[...]
        vs = _dot(xs_s[prv], wv_ref[...], ((1,), (1,))) + nf * bv_ref[...]
        vys = _dot(ys_s[prv], wvy_ref[...], ((1,), (1,))) + nf * bvy_ref[...]
        mats_x_s[...] = jnp.concatenate([matx, maty], axis=1)
        mats_y_s[...] = jnp.concatenate([maty, matx], axis=1)
        vs_s[...] = vs
        vys_s[...] = vys
        kp_s[:, 0:1] = ksT + EPS
        kp_s[:, 1:2] = kysT + EPS

    @pl.when(g > 0)
    def _phase_b():
        rows = pl.ds(j * bn, bn)
        qx = qx_s[prv, rows, :]
        qy = qy_s[prv, rows, :]
        tq = _dot(qx, kp_s[...], ((1,), (0,)))
        tqy = _dot(qy, kp_s[...], ((1,), (0,)))
        ax1 = s_ref[0] / (nf + tq[:, 0:1])
        ax2 = s_ref[1] / (nf + tq[:, 1:2])
        ay1 = s_ref[2] / (nf + tqy[:, 1:2])
        ay2 = s_ref[3] / (nf + tqy[:, 0:1])
        qmx = _dot(qx, mats_x_s[...], ((1,), (0,)))
        qmy = _dot(qy, mats_y_s[...], ((1,), (0,)))
        c = qmx.shape[1] // 2
        vs = vs_s[...]
        vys = vys_s[...]
        fx_ref[0] = ax1 * (vs + qmx[:, 0:c]) + ax2 * (vs + qmx[:, c:2 * c])
        fy_ref[0] = ay1 * (vys + qmy[:, 0:c]) + ay2 * (vys + qmy[:, c:2 * c])


def _run(x, y, Wq, bq, Wk, bk, Wv, bv, Wqy, bqy, Wky, bky, Wvy, bvy,
         gamma, gamma_y, gamma_cx, gamma_cy, wx1, wx2, wy1, wy2,
         interpret=False):
    b, n, c = x.shape
    d = Wq.shape[0]
    bn = min(2048, n)
    nb = n // bn
    f32 = jnp.float32

    in_map = lambda g, j: (jnp.where(g < b, g, b - 1),
                           jnp.where(g < b, j, nb - 1), 0)
    out_map = lambda g, j: (jnp.maximum(g - 1, 0),
                            jnp.where(g > 0, j, 0), 0)
    row_in = pl.BlockSpec((1, bn, c), in_map)
    row_out = pl.BlockSpec((1, bn, c), out_map)
    w_spec = lambda r, cc: pl.BlockSpec((r, cc), lambda g, j: (0, 0))

    wqk = jnp.concatenate([Wq, Wk], axis=0)
    wqky = jnp.concatenate([Wqy, Wky], axis=0)
    bqk = jnp.concatenate([bq, bk]).reshape(1, 2 * d)
    bqky = jnp.concatenate([bqy, bky]).reshape(1, 2 * d)
    s = jnp.stack([gamma[0] * wx1, gamma_cx[0] * wx2,
                   gamma_y[0] * wy1, gamma_cy[0] * wy2]).astype(f32)

    fx, fy = pl.pallas_call(
        functools.partial(_fused_kernel, b, nb, bn, d, n),
        grid=(b + 1, nb),
        in_specs=[
            row_in, row_in,
            w_spec(2 * d, c), w_spec(1, 2 * d),
            w_spec(2 * d, c), w_spec(1, 2 * d),
            w_spec(c, c), w_spec(1, c),
            w_spec(c, c), w_spec(1, c),
            pl.BlockSpec(memory_space=pltpu.SMEM),
        ],
        out_specs=[row_out, row_out],
        out_shape=[
            jax.ShapeDtypeStruct((b, n, c), f32),
            jax.ShapeDtypeStruct((b, n, c), f32),
        ],
        scratch_shapes=[
            pltpu.VMEM((2, n, d), f32),
            pltpu.VMEM((2, n, d), f32),
            pltpu.VMEM((2, d, c), f32),
            pltpu.VMEM((2, d, c), f32),
            pltpu.VMEM((2, 1, d), f32),
            pltpu.VMEM((2, 1, d), f32),
            pltpu.VMEM((2, 1, c), f32),
            pltpu.VMEM((2, 1, c), f32),
            pltpu.VMEM((d, 2 * c), f32),
            pltpu.VMEM((d, 2 * c), f32),
            pltpu.VMEM((1, c), f32),
            pltpu.VMEM((1, c), f32),
            pltpu.VMEM((d, 2), f32),
        ],
        compiler_params=pltpu.CompilerParams(
            dimension_semantics=("arbitrary", "arbitrary"),
            vmem_limit_bytes=56 * 1024 * 1024),
        name="linattn_fused",
        interpret=interpret,
    )(x, y, wqk, bqk, wqky, bqky,
      Wv, bv.reshape(1, c), Wvy, bvy.reshape(1, c), s)
    return fx, fy


def kernel(x, y, Wq, bq, Wk, bk, Wv, bv, Wqy, bqy, Wky, bky, Wvy, bvy,
           gamma, gamma_y, gamma_cx, gamma_cy, wx1, wx2, wy1, wy2):
    return _run(x, y, Wq, bq, Wk, bk, Wv, bv, Wqy, bqy, Wky, bky, Wvy, bvy,
                gamma, gamma_y, gamma_cx, gamma_cy, wx1, wx2, wy1, wy2)


# speedup vs baseline: 1.1848x; 1.0753x over previous
import functools

import jax
import jax.numpy as jnp
from jax.experimental import pallas as pl
from jax.experimental.pallas import tpu as pltpu

EPS = 1e-6


def _dot(a, b, dims):
    return jax.lax.dot_general(a, b, (dims, ((), ())),
                               preferred_element_type=jnp.float32)


def _fused_kernel(nbatch, nb, bn, d, n_rows,
                  x_ref, y_ref, wqk_ref, bqk_ref, wqky_ref, bqky_ref,
                  wv_ref, bv_ref, wvy_ref, bvy_ref, s_ref,
                  fx_ref, fy_ref,
                  qx_s, qy_s, m0x_s, m0y_s, ks_s, kys_s, xs_s, ys_s,
                  mats_x_s, mats_y_s, vs_s, vys_s, kp_s):
    g = pl.program_id(0)
    j = pl.program_id(1)
    cur = jax.lax.rem(g, 2)
    prv = 1 - cur
    nf = jnp.float32(nb * bn)

    @pl.when(jnp.logical_and(g < nbatch, j == 0))
    def _():
        m0x_s[cur] = jnp.zeros_like(m0x_s[cur])
        m0y_s[cur] = jnp.zeros_like(m0y_s[cur])
        ks_s[cur] = jnp.zeros_like(ks_s[cur])
        kys_s[cur] = jnp.zeros_like(kys_s[cur])
        xs_s[cur] = jnp.zeros_like(xs_s[cur])
        ys_s[cur] = jnp.zeros_like(ys_s[cur])

    @pl.when(g < nbatch)
    def _phase_a():
        rows = pl.ds(j * bn, bn)

        def stream(t_ref, wqk, bqk, q_s, m0_s, ks, xs):
            t = t_ref[0]
            wq = wqk[0:d, :]
            wk = wqk[d:2 * d, :]
            q = _dot(t, wq, ((1,), (1,))) + bqk[0:1, 0:d]
            k = _dot(t, wk, ((1,), (1,))) + bqk[0:1, d:2 * d]
            q = q * jax.lax.rsqrt(jnp.sum(q * q, axis=1, keepdims=True))
            k = k * jax.lax.rsqrt(jnp.sum(k * k, axis=1, keepdims=True))
            q_s[cur, rows, :] = q
            m0_s[cur] += _dot(k, t, ((0,), (0,)))
            ks[cur] += jnp.sum(k, axis=0, keepdims=True)
            xs[cur] += jnp.sum(t, axis=0, keepdims=True)

        stream(x_ref, wqk_ref, bqk_ref, qx_s, m0x_s, ks_s, xs_s)
        stream(y_ref, wqky_ref, bqky_ref, qy_s, m0y_s, kys_s, ys_s)

    @pl.when(jnp.logical_and(g > 0, j == 0))
    def _():
        ksT = jnp.transpose(ks_s[prv], (1, 0))
        kysT = jnp.transpose(kys_s[prv], (1, 0))
        matx = _dot(m0x_s[prv], wv_ref[...], ((1,), (1,))) \
            + ksT * bv_ref[...]
        maty = _dot(m0y_s[prv], wvy_ref[...], ((1,), (1,))) \
            + kysT * bvy_ref[...]
        vs = _dot(xs_s[prv], wv_ref[...], ((1,), (1,))) + nf * bv_ref[...]
        vys = _dot(ys_s[prv], wvy_ref[...], ((1,), (1,))) + nf * bvy_ref[...]
        mats_x_s[...] = jnp.concatenate([matx, maty], axis=1)
        mats_y_s[...] = jnp.concatenate([maty, matx], axis=1)
        vs_s[...] = vs
        vys_s[...] = vys
        kp_s[:, 0:1] = ksT + EPS
        kp_s[:, 1:2] = kysT + EPS

    @pl.when(g > 0)
    def _phase_b():
        rows = pl.ds(j * bn, bn)
        qx = qx_s[prv, rows, :]
        qy = qy_s[prv, rows, :]
        tq = _dot(qx, kp_s[...], ((1,), (0,)))
        tqy = _dot(qy, kp_s[...], ((1,), (0,)))
        ax1 = s_ref[0] / (nf + tq[:, 0:1])
        ax2 = s_ref[1] / (nf + tq[:, 1:2])
        ay1 = s_ref[2] / (nf + tqy[:, 1:2])
        ay2 = s_ref[3] / (nf + tqy[:, 0:1])
        qmx = _dot(qx, mats_x_s[...], ((1,), (0,)))
        qmy = _dot(qy, mats_y_s[...], ((1,), (0,)))
        c = qmx.shape[1] // 2
        vs = vs_s[...]
        vys = vys_s[...]
        fx_ref[0] = ax1 * (vs + qmx[:, 0:c]) + ax2 * (vs + qmx[:, c:2 * c])
        fy_ref[0] = ay1 * (vys + qmy[:, 0:c]) + ay2 * (vys + qmy[:, c:2 * c])


def _run(x, y, Wq, bq, Wk, bk, Wv, bv, Wqy, bqy, Wky, bky, Wvy, bvy,
         gamma, gamma_y, gamma_cx, gamma_cy, wx1, wx2, wy1, wy2,
         interpret=False):
    b, n, c = x.shape
    d = Wq.shape[0]
    bn = min(2048, n)
    nb = n // bn
    f32 = jnp.float32

    in_map = lambda g, j: (jnp.where(g < b, g, b - 1),
                           jnp.where(g < b, j, nb - 1), 0)
    out_map = lambda g, j: (jnp.maximum(g - 1, 0),
                            jnp.where(g > 0, j, 0), 0)
    row_in = pl.BlockSpec((1, bn, c), in_map)
    row_out = pl.BlockSpec((1, bn, c), out_map)
    w_spec = lambda r, cc: pl.BlockSpec((r, cc), lambda g, j: (0, 0))

    wqk = jnp.concatenate([Wq, Wk], axis=0)
    wqky = jnp.concatenate([Wqy, Wky], axis=0)
    bqk = jnp.concatenate([bq, bk]).reshape(1, 2 * d)
    bqky = jnp.concatenate([bqy, bky]).reshape(1, 2 * d)
    s = jnp.stack([gamma[0] * wx1, gamma_cx[0] * wx2,
                   gamma_y[0] * wy1, gamma_cy[0] * wy2]).astype(f32)

    fx, fy = pl.pallas_call(
        functools.partial(_fused_kernel, b, nb, bn, d, n),
        grid=(b + 1, nb),
        in_specs=[
            row_in, row_in,
            w_spec(2 * d, c), w_spec(1, 2 * d),
            w_spec(2 * d, c), w_spec(1, 2 * d),
            w_spec(c, c), w_spec(1, c),
            w_spec(c, c), w_spec(1, c),
            pl.BlockSpec(memory_space=pltpu.SMEM),
        ],
        out_specs=[row_out, row_out],
        out_shape=[
            jax.ShapeDtypeStruct((b, n, c), f32),
            jax.ShapeDtypeStruct((b, n, c), f32),
        ],
        scratch_shapes=[
            pltpu.VMEM((2, n, d), f32),
            pltpu.VMEM((2, n, d), f32),
            pltpu.VMEM((2, d, c), f32),
            pltpu.VMEM((2, d, c), f32),
            pltpu.VMEM((2, 1, d), f32),
            pltpu.VMEM((2, 1, d), f32),
            pltpu.VMEM((2, 1, c), f32),
            pltpu.VMEM((2, 1, c), f32),
            pltpu.VMEM((d, 2 * c), f32),
            pltpu.VMEM((d, 2 * c), f32),
            pltpu.VMEM((1, c), f32),
            pltpu.VMEM((1, c), f32),
            pltpu.VMEM((d, 2), f32),
        ],
        compiler_params=pltpu.CompilerParams(
            dimension_semantics=("arbitrary", "arbitrary"),
            vmem_limit_bytes=56 * 1024 * 1024,
            fuse_transposed_lhs_in_matmul=True),
        name="linattn_fused",
        interpret=interpret,
    )(x, y, wqk, bqk, wqky, bqky,
      Wv, bv.reshape(1, c), Wvy, bvy.reshape(1, c), s)
    return fx, fy


def kernel(x, y, Wq, bq, Wk, bk, Wv, bv, Wqy, bqy, Wky, bky, Wvy, bvy,
           gamma, gamma_y, gamma_cx, gamma_cy, wx1, wx2, wy1, wy2):
    return _run(x, y, Wq, bq, Wk, bk, Wv, bv, Wqy, bqy, Wky, bky, Wvy, bvy,
                gamma, gamma_y, gamma_cx, gamma_cy, wx1, wx2, wy1, wy2)
